# baseline (speedup 1.0000x reference)
import functools
import math

import jax
import jax.numpy as jnp
import numpy as np
from jax import lax
from jax.experimental import pallas as pl
from jax.experimental.pallas import tpu as pltpu
from jax.experimental.pallas import tpu_sc as plsc

MAX_LEN = 5000
DIM = 1024
N = 16384
B = 8
SCALE = math.sqrt(DIM)

LANES = 16
NUM_CHUNKS = N // LANES
NUM_CORES = 2
NUM_SUBCORES = 16
NUM_WORKERS = NUM_CORES * NUM_SUBCORES
CPW = NUM_CHUNKS // NUM_WORKERS
STATIC_ROWS = LANES
WIN_ROW = STATIC_ROWS
ZROW = 2 * LANES
PBUF_ROWS = 2 * LANES + 1


def _pe_table() -> np.ndarray:
    pe = np.zeros((MAX_LEN + 1, DIM), dtype=np.float32)
    position = np.arange(0, MAX_LEN, dtype=np.float32)[:, None]
    div_term = np.exp(
        np.arange(0, DIM, 2, dtype=np.float32) * -(math.log(10000.0) / DIM))
    pe[:MAX_LEN, 0::2] = np.sin(position * div_term)
    pe[:MAX_LEN, 1::2] = np.cos(position * div_term)
    return pe


_PE = _pe_table()


def _meta_body(mask_ref, out_ref):
    m = mask_ref[...]
    x = m
    for sh in (1, 2, 4, 8):
        x = x + jnp.concatenate(
            [jnp.zeros((NUM_CHUNKS, sh), jnp.int32), x[:, :-sh]], axis=1)
    rowm = x[:, LANES - 1:LANES]
    y = rowm
    for sh in (1, 2, 4, 8, 16, 32, 64, 128, 256, 512):
        y = y + jnp.concatenate(
            [jnp.zeros((sh, 1), jnp.int32), y[:-sh, :]], axis=0)
    k0 = y - rowm
    rank = k0 + x - 1
    col = lax.broadcasted_iota(jnp.int32, (NUM_CHUNKS, LANES), 1)
    bmod = col & 7
    cums, starts = [], []
    running = jnp.zeros((), jnp.int32)
    for b in range(B):
        sl_b = jnp.sum(jnp.where(bmod == b, m, 0))
        starts.append(running)
        running = running + sl_b
        cums.append(running)
    batchv = jnp.zeros((NUM_CHUNKS, LANES), jnp.int32)
    batch0 = jnp.zeros((NUM_CHUNKS, 1), jnp.int32)
    for b in range(B):
        batchv = batchv + (rank >= cums[b]).astype(jnp.int32)
        batch0 = batch0 + (k0 >= cums[b]).astype(jnp.int32)
    batchv = jnp.minimum(batchv, B - 1)
    batch0 = jnp.minimum(batch0, B - 1)
    startv = jnp.zeros((NUM_CHUNKS, LANES), jnp.int32)
    start0 = jnp.zeros((NUM_CHUNKS, 1), jnp.int32)
    for b in range(B):
        startv = startv + jnp.where(batchv == b, starts[b], 0)
        start0 = start0 + jnp.where(batch0 == b, starts[b], 0)
    pos = rank - startv
    pos0 = jnp.clip(k0 - start0, 0, MAX_LEN - LANES)
    active = m > 0
    same = batchv == batch0
    loff = jnp.where(active,
                     jnp.where(same, pos - pos0 + WIN_ROW, pos),
                     ZROW)
    out_ref[...] = jnp.concatenate(
        [jnp.broadcast_to(pos0, (NUM_CHUNKS, LANES)), loff], axis=1)


def _chunk_meta(mask_i32):
    return pl.pallas_call(
        _meta_body,
        out_shape=jax.ShapeDtypeStruct((NUM_CHUNKS, 2 * LANES), jnp.int32),
    )(mask_i32)


CHUNK_ELEMS = LANES * DIM
UNROLL = 8


def _sc_body(emb_hbm, meta_hbm, pe_hbm, out_hbm,
             eb0, eb1, pb0, pb1, mb0, mb1,
             es0, es1, gs0, gs1, ss0, ss1, ms0, ms1):
    eb, pb, mb = (eb0, eb1), (pb0, pb1), (mb0, mb1)
    es, gs, ss, ms = (es0, es1), (gs0, gs1), (ss0, ss1), (ms0, ms1)
    wid = lax.axis_index("s") * NUM_CORES + lax.axis_index("c")
    gbase = wid * CPW

    for i in (0, 1):
        pltpu.sync_copy(pe_hbm.at[pl.ds(0, STATIC_ROWS * DIM)],
                        pb[i].at[pl.ds(0, STATIC_ROWS * DIM)])
        pltpu.sync_copy(pe_hbm.at[pl.ds(MAX_LEN * DIM, DIM)],
                        pb[i].at[pl.ds(ZROW * DIM, DIM)])

    def start_meta(ch, b):
        pltpu.async_copy(meta_hbm.at[pl.ds((gbase + ch) * 2 * LANES,
                                           2 * LANES)], mb[b], ms[b])

    def wait_meta(ch, b):
        pltpu.make_async_copy(meta_hbm.at[pl.ds(0, 2 * LANES)],
                              mb[b], ms[b]).wait()

    def start_emb(ch, b):
        e0 = (gbase + ch) * CHUNK_ELEMS
        pltpu.async_copy(emb_hbm.at[pl.ds(e0, CHUNK_ELEMS)], eb[b], es[b])

    def wait_emb(b):
        pltpu.make_async_copy(emb_hbm.at[pl.ds(0, CHUNK_ELEMS)],
                              eb[b], es[b]).wait()

    def start_window(b):
        pos0 = jnp.max(mb[b][pl.ds(0, LANES)])
        pltpu.async_copy(pe_hbm.at[pl.ds(pos0 * DIM, LANES * DIM)],
                         pb[b].at[pl.ds(WIN_ROW * DIM, LANES * DIM)], gs[b])

    def wait_window(b):
        pltpu.make_async_copy(pe_hbm.at[pl.ds(0, LANES * DIM)],
                              pb[b].at[pl.ds(WIN_ROW * DIM, LANES * DIM)],
                              gs[b]).wait()

    def wait_store(ch, b):
        e0 = (gbase + ch) * CHUNK_ELEMS
        pltpu.make_async_copy(eb[b], out_hbm.at[pl.ds(e0, CHUNK_ELEMS)],
                              ss[b]).wait()

    def step(ch, b):
        nb = 1 - b

        @pl.when(ch >= 1)
        def _():
            wait_store(ch - 1, nb)

        @pl.when(ch + 1 < CPW)
        def _():
            start_emb(ch + 1, nb)
            wait_meta(ch + 1, nb)
            start_window(nb)

        wait_emb(b)
        wait_window(b)

        lvec = mb[b][pl.ds(LANES, LANES)]
        riota = lax.broadcasted_iota(jnp.int32, (LANES,), 0)
        pe_idx0 = lvec * DIM
        e_idx0 = riota * DIM

        def col_fn(i, carry):
            pe_idx, e_idx = carry
            for k in range(UNROLL):
                pk = pe_idx + k
                ek = e_idx + k
                pe_col = plsc.load_gather(pb[b], [pk])
                emb_col = plsc.load_gather(eb[b], [ek])
                plsc.store_scatter(eb[b], [ek], emb_col * SCALE + pe_col)
            return pe_idx + UNROLL, e_idx + UNROLL

        lax.fori_loop(0, DIM // UNROLL, col_fn, (pe_idx0, e_idx0))

        e0 = (gbase + ch) * CHUNK_ELEMS
        pltpu.async_copy(eb[b], out_hbm.at[pl.ds(e0, CHUNK_ELEMS)], ss[b])

        @pl.when(ch + 2 < CPW)
        def _():
            start_meta(ch + 2, b)

    start_meta(0, 0)
    start_meta(1, 1)
    start_emb(0, 0)
    wait_meta(0, 0)
    start_window(0)

    def pair_fn(pair, carry):
        step(2 * pair, 0)
        step(2 * pair + 1, 1)
        return carry

    lax.fori_loop(0, CPW // 2, pair_fn, 0)
    wait_store(CPW - 1, 1)


@functools.cache
def _sc_apply():
    return pl.kernel(
        _sc_body,
        mesh=plsc.VectorSubcoreMesh(core_axis_name="c", subcore_axis_name="s"),
        compiler_params=pltpu.CompilerParams(needs_layout_passes=False),
        out_type=jax.ShapeDtypeStruct((N * DIM,), jnp.float32),
        scratch_types=[
            pltpu.VMEM((LANES * DIM,), jnp.float32),
            pltpu.VMEM((LANES * DIM,), jnp.float32),
            pltpu.VMEM((PBUF_ROWS * DIM,), jnp.float32),
            pltpu.VMEM((PBUF_ROWS * DIM,), jnp.float32),
            pltpu.VMEM((2 * LANES,), jnp.int32),
            pltpu.VMEM((2 * LANES,), jnp.int32),
            pltpu.SemaphoreType.DMA, pltpu.SemaphoreType.DMA,
            pltpu.SemaphoreType.DMA, pltpu.SemaphoreType.DMA,
            pltpu.SemaphoreType.DMA, pltpu.SemaphoreType.DMA,
            pltpu.SemaphoreType.DMA, pltpu.SemaphoreType.DMA,
        ],
    )


def kernel(emb, position_mask):
    mask_i32 = position_mask.reshape(NUM_CHUNKS, LANES).astype(jnp.int32)
    meta = _chunk_meta(mask_i32).reshape(-1)
    emb_flat = emb.reshape(-1)
    out_flat = _sc_apply()(emb_flat, meta, jnp.asarray(_PE).reshape(-1))
    return out_flat.reshape(emb.shape)


# speedup vs baseline: 3.2611x; 3.2611x over previous
import functools
import math

import jax
import jax.numpy as jnp
import numpy as np
from jax import lax
from jax.experimental import pallas as pl
from jax.experimental.pallas import tpu as pltpu
from jax.experimental.pallas import tpu_sc as plsc

MAX_LEN = 5000
DIM = 1024
N = 16384
B = 8
SCALE = math.sqrt(DIM)

LANES = 16
NUM_CHUNKS = N // LANES
NUM_CORES = 2
NUM_SUBCORES = 16
NUM_WORKERS = NUM_CORES * NUM_SUBCORES
CPW = NUM_CHUNKS // NUM_WORKERS
STATIC_ROWS = LANES
WIN_ROW = STATIC_ROWS
ZROW = 2 * LANES
PBUF_ROWS = 2 * LANES + 1


def _pe_table() -> np.ndarray:
    pe = np.zeros((MAX_LEN + 1, DIM), dtype=np.float32)
    position = np.arange(0, MAX_LEN, dtype=np.float32)[:, None]
    div_term = np.exp(
        np.arange(0, DIM, 2, dtype=np.float32) * -(math.log(10000.0) / DIM))
    pe[:MAX_LEN, 0::2] = np.sin(position * div_term)
    pe[:MAX_LEN, 1::2] = np.cos(position * div_term)
    return pe


_PE = _pe_table()


def _meta_body(mask_ref, out_ref):
    m = mask_ref[...]
    x = m
    for sh in (1, 2, 4, 8):
        x = x + jnp.concatenate(
            [jnp.zeros((NUM_CHUNKS, sh), jnp.int32), x[:, :-sh]], axis=1)
    rowm = x[:, LANES - 1:LANES]
    y = rowm
    for sh in (1, 2, 4, 8, 16, 32, 64, 128, 256, 512):
        y = y + jnp.concatenate(
            [jnp.zeros((sh, 1), jnp.int32), y[:-sh, :]], axis=0)
    k0 = y - rowm
    rank = k0 + x - 1
    col = lax.broadcasted_iota(jnp.int32, (NUM_CHUNKS, LANES), 1)
    bmod = col & 7
    cums, starts = [], []
    running = jnp.zeros((), jnp.int32)
    for b in range(B):
        sl_b = jnp.sum(jnp.where(bmod == b, m, 0))
        starts.append(running)
        running = running + sl_b
        cums.append(running)
    batchv = jnp.zeros((NUM_CHUNKS, LANES), jnp.int32)
    batch0 = jnp.zeros((NUM_CHUNKS, 1), jnp.int32)
    for b in range(B):
        batchv = batchv + (rank >= cums[b]).astype(jnp.int32)
        batch0 = batch0 + (k0 >= cums[b]).astype(jnp.int32)
    batchv = jnp.minimum(batchv, B - 1)
    batch0 = jnp.minimum(batch0, B - 1)
    startv = jnp.zeros((NUM_CHUNKS, LANES), jnp.int32)
    start0 = jnp.zeros((NUM_CHUNKS, 1), jnp.int32)
    for b in range(B):
        startv = startv + jnp.where(batchv == b, starts[b], 0)
        start0 = start0 + jnp.where(batch0 == b, starts[b], 0)
    pos = rank - startv
    pos0 = jnp.clip(k0 - start0, 0, MAX_LEN - LANES)
    active = m > 0
    same = batchv == batch0
    loff = jnp.where(active,
                     jnp.where(same, pos - pos0 + WIN_ROW, pos),
                     ZROW)
    out_ref[...] = jnp.concatenate(
        [jnp.broadcast_to(pos0, (NUM_CHUNKS, LANES)), loff], axis=1)


def _chunk_meta(mask_i32):
    return pl.pallas_call(
        _meta_body,
        out_shape=jax.ShapeDtypeStruct((NUM_CHUNKS, 2 * LANES), jnp.int32),
    )(mask_i32)


CHUNK_ELEMS = LANES * DIM
UNROLL = 8


def _sc_body(emb_hbm, meta_hbm, pe_hbm, out_hbm,
             eb0, eb1, pb0, pb1, mb0, mb1,
             es0, es1, gs0, gs1, ss0, ss1, ms0, ms1):
    eb, pb, mb = (eb0, eb1), (pb0, pb1), (mb0, mb1)
    es, gs, ss, ms = (es0, es1), (gs0, gs1), (ss0, ss1), (ms0, ms1)
    wid = lax.axis_index("s") * NUM_CORES + lax.axis_index("c")
    gbase = wid * CPW

    for i in (0, 1):
        pltpu.sync_copy(pe_hbm.at[pl.ds(0, STATIC_ROWS * DIM)],
                        pb[i].at[pl.ds(0, STATIC_ROWS * DIM)])
        pltpu.sync_copy(pe_hbm.at[pl.ds(MAX_LEN * DIM, DIM)],
                        pb[i].at[pl.ds(ZROW * DIM, DIM)])

    def start_meta(ch, b):
        pltpu.async_copy(meta_hbm.at[pl.ds((gbase + ch) * 2 * LANES,
                                           2 * LANES)], mb[b], ms[b])

    def wait_meta(ch, b):
        pltpu.make_async_copy(meta_hbm.at[pl.ds(0, 2 * LANES)],
                              mb[b], ms[b]).wait()

    def start_emb(ch, b):
        e0 = (gbase + ch) * CHUNK_ELEMS
        pltpu.async_copy(emb_hbm.at[pl.ds(e0, CHUNK_ELEMS)], eb[b], es[b])

    def wait_emb(b):
        pltpu.make_async_copy(emb_hbm.at[pl.ds(0, CHUNK_ELEMS)],
                              eb[b], es[b]).wait()

    def start_window(b):
        pos0 = jnp.max(mb[b][pl.ds(0, LANES)])
        pltpu.async_copy(pe_hbm.at[pl.ds(pos0 * DIM, LANES * DIM)],
                         pb[b].at[pl.ds(WIN_ROW * DIM, LANES * DIM)], gs[b])

    def wait_window(b):
        pltpu.make_async_copy(pe_hbm.at[pl.ds(0, LANES * DIM)],
                              pb[b].at[pl.ds(WIN_ROW * DIM, LANES * DIM)],
                              gs[b]).wait()

    def wait_store(ch, b):
        e0 = (gbase + ch) * CHUNK_ELEMS
        pltpu.make_async_copy(eb[b], out_hbm.at[pl.ds(e0, CHUNK_ELEMS)],
                              ss[b]).wait()

    def step(ch, b):
        nb = 1 - b

        @pl.when(ch >= 1)
        def _():
            wait_store(ch - 1, nb)

        @pl.when(ch + 1 < CPW)
        def _():
            start_emb(ch + 1, nb)
            wait_meta(ch + 1, nb)
            start_window(nb)

        wait_emb(b)
        wait_window(b)

        lvec = mb[b][pl.ds(LANES, LANES)]
        riota = lax.broadcasted_iota(jnp.int32, (LANES,), 0)

        def row_fn(r, carry):
            loff_r = jnp.max(jnp.where(riota == r, lvec, 0))
            pbase = loff_r * DIM
            ebase = r * DIM
            for c0 in range(0, DIM, LANES):
                e = eb[b][pl.ds(ebase + c0, LANES)]
                p = pb[b][pl.ds(pbase + c0, LANES)]
                eb[b][pl.ds(ebase + c0, LANES)] = e * SCALE + p
            return carry

        lax.fori_loop(0, LANES, row_fn, 0)

        e0 = (gbase + ch) * CHUNK_ELEMS
        pltpu.async_copy(eb[b], out_hbm.at[pl.ds(e0, CHUNK_ELEMS)], ss[b])

        @pl.when(ch + 2 < CPW)
        def _():
            start_meta(ch + 2, b)

    start_meta(0, 0)
    start_meta(1, 1)
    start_emb(0, 0)
    wait_meta(0, 0)
    start_window(0)

    def pair_fn(pair, carry):
        step(2 * pair, 0)
        step(2 * pair + 1, 1)
        return carry

    lax.fori_loop(0, CPW // 2, pair_fn, 0)
    wait_store(CPW - 1, 1)


@functools.cache
def _sc_apply():
    return pl.kernel(
        _sc_body,
        mesh=plsc.VectorSubcoreMesh(core_axis_name="c", subcore_axis_name="s"),
        compiler_params=pltpu.CompilerParams(needs_layout_passes=False),
        out_type=jax.ShapeDtypeStruct((N * DIM,), jnp.float32),
        scratch_types=[
            pltpu.VMEM((LANES * DIM,), jnp.float32),
            pltpu.VMEM((LANES * DIM,), jnp.float32),
            pltpu.VMEM((PBUF_ROWS * DIM,), jnp.float32),
            pltpu.VMEM((PBUF_ROWS * DIM,), jnp.float32),
            pltpu.VMEM((2 * LANES,), jnp.int32),
            pltpu.VMEM((2 * LANES,), jnp.int32),
            pltpu.SemaphoreType.DMA, pltpu.SemaphoreType.DMA,
            pltpu.SemaphoreType.DMA, pltpu.SemaphoreType.DMA,
            pltpu.SemaphoreType.DMA, pltpu.SemaphoreType.DMA,
            pltpu.SemaphoreType.DMA, pltpu.SemaphoreType.DMA,
        ],
    )


def kernel(emb, position_mask):
    mask_i32 = position_mask.reshape(NUM_CHUNKS, LANES).astype(jnp.int32)
    meta = _chunk_meta(mask_i32).reshape(-1)
    emb_flat = emb.reshape(-1)
    out_flat = _sc_apply()(emb_flat, meta, jnp.asarray(_PE).reshape(-1))
    return out_flat.reshape(emb.shape)


# speedup vs baseline: 4.0078x; 1.2290x over previous
import functools
import math

import jax
import jax.numpy as jnp
import numpy as np
from jax import lax
from jax.experimental import pallas as pl
from jax.experimental.pallas import tpu as pltpu
from jax.experimental.pallas import tpu_sc as plsc

MAX_LEN = 5000
DIM = 1024
N = 16384
B = 8
SCALE = math.sqrt(DIM)

LANES = 16
NUM_CHUNKS = N // LANES
NUM_CORES = 2
NUM_SUBCORES = 16
NUM_WORKERS = NUM_CORES * NUM_SUBCORES
CPW = 10
STATIC_ROWS = LANES
WIN_ROW = STATIC_ROWS
ZROW = 2 * LANES
PBUF_ROWS = 2 * LANES + 1
PE_ROWS = 2047

BLK = 512
SC_BLOCKS = 10
TC_BLOCKS = N // BLK - SC_BLOCKS
SC_CHUNKS = SC_BLOCKS * BLK // LANES
PE_TC_ROWS = 2048


def _pe_table() -> np.ndarray:
    pe = np.zeros((MAX_LEN + 1, DIM), dtype=np.float32)
    position = np.arange(0, MAX_LEN, dtype=np.float32)[:, None]
    div_term = np.exp(
        np.arange(0, DIM, 2, dtype=np.float32) * -(math.log(10000.0) / DIM))
    pe[:MAX_LEN, 0::2] = np.sin(position * div_term)
    pe[:MAX_LEN, 1::2] = np.cos(position * div_term)
    return pe


_PE = _pe_table()


def _meta_body(mask_ref, out_ref, pidx_ref):
    m = mask_ref[...]
    x = m
    for sh in (1, 2, 4, 8):
        x = x + jnp.concatenate(
            [jnp.zeros((NUM_CHUNKS, sh), jnp.int32), x[:, :-sh]], axis=1)
    rowm = x[:, LANES - 1:LANES]
    y = rowm
    for sh in (1, 2, 4, 8, 16, 32, 64, 128, 256, 512):
        y = y + jnp.concatenate(
            [jnp.zeros((sh, 1), jnp.int32), y[:-sh, :]], axis=0)
    k0 = y - rowm
    rank = k0 + x - 1
    col = lax.broadcasted_iota(jnp.int32, (NUM_CHUNKS, LANES), 1)
    bmod = col & 7
    cums, starts = [], []
    running = jnp.zeros((), jnp.int32)
    for b in range(B):
        sl_b = jnp.sum(jnp.where(bmod == b, m, 0))
        starts.append(running)
        running = running + sl_b
        cums.append(running)
    batchv = jnp.zeros((NUM_CHUNKS, LANES), jnp.int32)
    batch0 = jnp.zeros((NUM_CHUNKS, 1), jnp.int32)
    for b in range(B):
        batchv = batchv + (rank >= cums[b]).astype(jnp.int32)
        batch0 = batch0 + (k0 >= cums[b]).astype(jnp.int32)
    batchv = jnp.minimum(batchv, B - 1)
    batch0 = jnp.minimum(batch0, B - 1)
    startv = jnp.zeros((NUM_CHUNKS, LANES), jnp.int32)
    start0 = jnp.zeros((NUM_CHUNKS, 1), jnp.int32)
    for b in range(B):
        startv = startv + jnp.where(batchv == b, starts[b], 0)
        start0 = start0 + jnp.where(batch0 == b, starts[b], 0)
    pos = rank - startv
    w0 = jnp.clip(k0 - start0, 1, PE_ROWS + 1 - LANES)
    active = m > 0
    loff = jnp.where(active,
                     jnp.where(pos >= w0, pos - w0 + WIN_ROW, pos),
                     ZROW)
    out_ref[...] = jnp.concatenate(
        [jnp.broadcast_to(w0, (NUM_CHUNKS, LANES)), loff], axis=1)
    pidx_ref[...] = jnp.where(active, pos, PE_TC_ROWS)


def _chunk_meta(mask_i32):
    return pl.pallas_call(
        _meta_body,
        out_shape=(
            jax.ShapeDtypeStruct((NUM_CHUNKS, 2 * LANES), jnp.int32),
            jax.ShapeDtypeStruct((NUM_CHUNKS, LANES), jnp.int32),
        ),
    )(mask_i32)


CHUNK_ELEMS = LANES * DIM
UNROLL = 8


def _sc_body(emb_hbm, meta_hbm, pe_hbm, out_hbm,
             eb0, eb1, pb0, pb1, mb0, mb1,
             es0, es1, gs0, gs1, ss0, ss1, ms0, ms1):
    eb, pb, mb = (eb0, eb1), (pb0, pb1), (mb0, mb1)
    es, gs, ss, ms = (es0, es1), (gs0, gs1), (ss0, ss1), (ms0, ms1)
    wid = lax.axis_index("s") * NUM_CORES + lax.axis_index("c")
    gbase = wid * CPW

    for i in (0, 1):
        pltpu.sync_copy(pe_hbm.at[pl.ds(0, STATIC_ROWS * DIM)],
                        pb[i].at[pl.ds(0, STATIC_ROWS * DIM)])
        pltpu.sync_copy(pe_hbm.at[pl.ds(MAX_LEN * DIM, DIM)],
                        pb[i].at[pl.ds(ZROW * DIM, DIM)])

    def start_meta(ch, b):
        pltpu.async_copy(meta_hbm.at[pl.ds((gbase + ch) * 2 * LANES,
                                           2 * LANES)], mb[b], ms[b])

    def wait_meta(ch, b):
        pltpu.make_async_copy(meta_hbm.at[pl.ds(0, 2 * LANES)],
                              mb[b], ms[b]).wait()

    def start_emb(ch, b):
        e0 = (gbase + ch) * CHUNK_ELEMS
        pltpu.async_copy(emb_hbm.at[pl.ds(e0, CHUNK_ELEMS)], eb[b], es[b])

    def wait_emb(b):
        pltpu.make_async_copy(emb_hbm.at[pl.ds(0, CHUNK_ELEMS)],
                              eb[b], es[b]).wait()

    def start_window(b):
        w0 = jnp.max(mb[b][pl.ds(0, LANES)])
        pltpu.async_copy(pe_hbm.at[pl.ds(w0 * DIM, LANES * DIM)],
                         pb[b].at[pl.ds(WIN_ROW * DIM, LANES * DIM)], gs[b])

    def wait_window(b):
        pltpu.make_async_copy(pe_hbm.at[pl.ds(0, LANES * DIM)],
                              pb[b].at[pl.ds(WIN_ROW * DIM, LANES * DIM)],
                              gs[b]).wait()

    def wait_store(ch, b):
        e0 = (gbase + ch) * CHUNK_ELEMS
        pltpu.make_async_copy(eb[b], out_hbm.at[pl.ds(e0, CHUNK_ELEMS)],
                              ss[b]).wait()

    def step(ch, b):
        nb = 1 - b

        @pl.when(ch >= 1)
        def _():
            wait_store(ch - 1, nb)

        @pl.when(ch + 1 < CPW)
        def _():
            start_emb(ch + 1, nb)
            wait_meta(ch + 1, nb)
            start_window(nb)

        wait_emb(b)
        wait_window(b)

        lvec = mb[b][pl.ds(LANES, LANES)]
        riota = lax.broadcasted_iota(jnp.int32, (LANES,), 0)

        def row_fn(r, carry):
            loff_r = jnp.max(jnp.where(riota == r, lvec, 0))
            pbase = loff_r * DIM
            ebase = r * DIM
            for c0 in range(0, DIM, LANES):
                e = eb[b][pl.ds(ebase + c0, LANES)]
                p = pb[b][pl.ds(pbase + c0, LANES)]
                eb[b][pl.ds(ebase + c0, LANES)] = e * SCALE + p
            return carry

        lax.fori_loop(0, LANES, row_fn, 0)

        e0 = (gbase + ch) * CHUNK_ELEMS
        pltpu.async_copy(eb[b], out_hbm.at[pl.ds(e0, CHUNK_ELEMS)], ss[b])

        @pl.when(ch + 2 < CPW)
        def _():
            start_meta(ch + 2, b)

    start_meta(0, 0)
    start_meta(1, 1)
    start_emb(0, 0)
    wait_meta(0, 0)
    start_window(0)

    def pair_fn(pair, carry):
        step(2 * pair, 0)
        step(2 * pair + 1, 1)
        return carry

    lax.fori_loop(0, CPW // 2, pair_fn, 0)
    wait_store(CPW - 1, 1)


@functools.cache
def _sc_apply():
    return pl.kernel(
        _sc_body,
        mesh=plsc.VectorSubcoreMesh(core_axis_name="c", subcore_axis_name="s"),
        compiler_params=pltpu.CompilerParams(needs_layout_passes=False),
        out_type=jax.ShapeDtypeStruct((N * DIM,), jnp.float32),
        scratch_types=[
            pltpu.VMEM((LANES * DIM,), jnp.float32),
            pltpu.VMEM((LANES * DIM,), jnp.float32),
            pltpu.VMEM((PBUF_ROWS * DIM,), jnp.float32),
            pltpu.VMEM((PBUF_ROWS * DIM,), jnp.float32),
            pltpu.VMEM((2 * LANES,), jnp.int32),
            pltpu.VMEM((2 * LANES,), jnp.int32),
            pltpu.SemaphoreType.DMA, pltpu.SemaphoreType.DMA,
            pltpu.SemaphoreType.DMA, pltpu.SemaphoreType.DMA,
            pltpu.SemaphoreType.DMA, pltpu.SemaphoreType.DMA,
            pltpu.SemaphoreType.DMA, pltpu.SemaphoreType.DMA,
        ],
    )


def _tc_body(out_in_ref, emb_ref, pidx_ref, pe_ref, out_ref):
    del out_in_ref
    pidx = pidx_ref[0, 0, :].reshape(BLK, 1)
    iota = lax.broadcasted_iota(jnp.int32, (BLK, PE_TC_ROWS), 1)
    onehot = (pidx == iota).astype(jnp.bfloat16)
    pe_sel = lax.dot_general(onehot, pe_ref[...], (((1,), (0,)), ((), ())),
                             preferred_element_type=jnp.float32)
    out_ref[...] = emb_ref[...] * SCALE + pe_sel


@functools.cache
def _tc_apply():
    return pl.pallas_call(
        _tc_body,
        grid=(TC_BLOCKS,),
        in_specs=[
            pl.BlockSpec((8, 128), lambda j: (0, 0)),
            pl.BlockSpec((BLK, DIM), lambda j: (SC_BLOCKS + j, 0)),
            pl.BlockSpec((1, 1, BLK), lambda j: (SC_BLOCKS + j, 0, 0)),
            pl.BlockSpec((PE_TC_ROWS, DIM), lambda j: (0, 0)),
        ],
        out_specs=pl.BlockSpec((BLK, DIM), lambda j: (SC_BLOCKS + j, 0)),
        out_shape=jax.ShapeDtypeStruct((N, DIM), jnp.float32),
        input_output_aliases={0: 0},
    )


def kernel(emb, position_mask):
    mask_i32 = position_mask.reshape(NUM_CHUNKS, LANES).astype(jnp.int32)
    meta, pidx = _chunk_meta(mask_i32)
    emb_flat = emb.reshape(-1)
    out_flat = _sc_apply()(emb_flat, meta.reshape(-1),
                           jnp.asarray(_PE).reshape(-1))
    out_tc = _tc_apply()(
        out_flat.reshape(N, DIM),
        emb.reshape(N, DIM),
        pidx.reshape(N // BLK, 1, BLK),
        jnp.asarray(_PE[:PE_TC_ROWS].astype(np.dtype(jnp.bfloat16))),
    )
    return out_tc.reshape(emb.shape)


# speedup vs baseline: 6.5883x; 1.6439x over previous
import functools
import math

import jax
import jax.numpy as jnp
import numpy as np
from jax import lax
from jax.experimental import pallas as pl
from jax.experimental.pallas import tpu as pltpu
from jax.experimental.pallas import tpu_sc as plsc

MAX_LEN = 5000
DIM = 1024
N = 16384
B = 8
SCALE = math.sqrt(DIM)

LANES = 16
NUM_CHUNKS = N // LANES
NUM_CORES = 2
NUM_SUBCORES = 16
NUM_WORKERS = NUM_CORES * NUM_SUBCORES
CPW = 10
STATIC_ROWS = LANES
WIN_ROW = STATIC_ROWS
ZROW = 2 * LANES
PBUF_ROWS = 2 * LANES + 1
PE_ROWS = 2047

BLK = 512
SC_BLOCKS = 10
TC_BLOCKS = N // BLK - SC_BLOCKS
SC_CHUNKS = SC_BLOCKS * BLK // LANES
PE_TC_ROWS = 2048


def _pe_table() -> np.ndarray:
    pe = np.zeros((MAX_LEN + 1, DIM), dtype=np.float32)
    position = np.arange(0, MAX_LEN, dtype=np.float32)[:, None]
    div_term = np.exp(
        np.arange(0, DIM, 2, dtype=np.float32) * -(math.log(10000.0) / DIM))
    pe[:MAX_LEN, 0::2] = np.sin(position * div_term)
    pe[:MAX_LEN, 1::2] = np.cos(position * div_term)
    return pe


_PE = _pe_table()


def _meta_body(mask_ref, out_ref, pidx_ref):
    m = mask_ref[...]
    x = m
    for sh in (1, 2, 4, 8):
        x = x + jnp.concatenate(
            [jnp.zeros((NUM_CHUNKS, sh), jnp.int32), x[:, :-sh]], axis=1)
    rowm = x[:, LANES - 1:LANES]
    y = rowm
    for sh in (1, 2, 4, 8, 16, 32, 64, 128, 256, 512):
        y = y + jnp.concatenate(
            [jnp.zeros((sh, 1), jnp.int32), y[:-sh, :]], axis=0)
    k0 = y - rowm
    rank = k0 + x - 1
    col = lax.broadcasted_iota(jnp.int32, (NUM_CHUNKS, LANES), 1)
    bmod = col & 7
    cums, starts = [], []
    running = jnp.zeros((), jnp.int32)
    for b in range(B):
        sl_b = jnp.sum(jnp.where(bmod == b, m, 0))
        starts.append(running)
        running = running + sl_b
        cums.append(running)
    batchv = jnp.zeros((NUM_CHUNKS, LANES), jnp.int32)
    batch0 = jnp.zeros((NUM_CHUNKS, 1), jnp.int32)
    for b in range(B):
        batchv = batchv + (rank >= cums[b]).astype(jnp.int32)
        batch0 = batch0 + (k0 >= cums[b]).astype(jnp.int32)
    batchv = jnp.minimum(batchv, B - 1)
    batch0 = jnp.minimum(batch0, B - 1)
    startv = jnp.zeros((NUM_CHUNKS, LANES), jnp.int32)
    start0 = jnp.zeros((NUM_CHUNKS, 1), jnp.int32)
    for b in range(B):
        startv = startv + jnp.where(batchv == b, starts[b], 0)
        start0 = start0 + jnp.where(batch0 == b, starts[b], 0)
    pos = rank - startv
    w0 = jnp.clip(k0 - start0, 1, PE_ROWS + 1 - LANES)
    active = m > 0
    loff = jnp.where(active,
                     jnp.where(pos >= w0, pos - w0 + WIN_ROW, pos),
                     ZROW)
    out_ref[...] = jnp.concatenate(
        [jnp.broadcast_to(w0, (NUM_CHUNKS, LANES)), loff], axis=1)
    pidx_ref[...] = jnp.where(active, pos, PE_TC_ROWS)


def _chunk_meta(mask_i32):
    return pl.pallas_call(
        _meta_body,
        out_shape=(
            jax.ShapeDtypeStruct((NUM_CHUNKS, 2 * LANES), jnp.int32),
            jax.ShapeDtypeStruct((NUM_CHUNKS, LANES), jnp.int32),
        ),
    )(mask_i32)


CHUNK_ELEMS = LANES * DIM
UNROLL = 8


def _sc_body(emb_hbm, meta_hbm, pe_hbm, out_hbm,
             eb0, eb1, pb0, pb1, mb0, mb1,
             es0, es1, gs0, gs1, ss0, ss1, ms0, ms1):
    eb, pb, mb = (eb0, eb1), (pb0, pb1), (mb0, mb1)
    es, gs, ss, ms = (es0, es1), (gs0, gs1), (ss0, ss1), (ms0, ms1)
    wid = lax.axis_index("s") * NUM_CORES + lax.axis_index("c")
    gbase = wid * CPW

    for i in (0, 1):
        pltpu.sync_copy(pe_hbm.at[pl.ds(0, STATIC_ROWS * DIM)],
                        pb[i].at[pl.ds(0, STATIC_ROWS * DIM)])
        pltpu.sync_copy(pe_hbm.at[pl.ds(MAX_LEN * DIM, DIM)],
                        pb[i].at[pl.ds(ZROW * DIM, DIM)])

    def start_meta(ch, b):
        pltpu.async_copy(meta_hbm.at[pl.ds((gbase + ch) * 2 * LANES,
                                           2 * LANES)], mb[b], ms[b])

    def wait_meta(ch, b):
        pltpu.make_async_copy(meta_hbm.at[pl.ds(0, 2 * LANES)],
                              mb[b], ms[b]).wait()

    def start_emb(ch, b):
        r0 = (gbase + ch) * LANES
        pltpu.async_copy(emb_hbm.at[pl.ds(r0, LANES)], eb[b], es[b])

    def wait_emb(b):
        pltpu.make_async_copy(emb_hbm.at[pl.ds(0, LANES)],
                              eb[b], es[b]).wait()

    def start_window(b):
        w0 = jnp.max(mb[b][pl.ds(0, LANES)])
        pltpu.async_copy(pe_hbm.at[pl.ds(w0 * DIM, LANES * DIM)],
                         pb[b].at[pl.ds(WIN_ROW * DIM, LANES * DIM)], gs[b])

    def wait_window(b):
        pltpu.make_async_copy(pe_hbm.at[pl.ds(0, LANES * DIM)],
                              pb[b].at[pl.ds(WIN_ROW * DIM, LANES * DIM)],
                              gs[b]).wait()

    def wait_store(ch, b):
        r0 = (gbase + ch) * LANES
        pltpu.make_async_copy(eb[b], out_hbm.at[pl.ds(r0, LANES)],
                              ss[b]).wait()

    def step(ch, b):
        nb = 1 - b

        @pl.when(ch >= 1)
        def _():
            wait_store(ch - 1, nb)

        @pl.when(ch + 1 < CPW)
        def _():
            start_emb(ch + 1, nb)
            wait_meta(ch + 1, nb)
            start_window(nb)

        wait_emb(b)
        wait_window(b)

        lvec = mb[b][pl.ds(LANES, LANES)]
        riota = lax.broadcasted_iota(jnp.int32, (LANES,), 0)

        def row_fn(r, carry):
            loff_r = jnp.max(jnp.where(riota == r, lvec, 0))
            pbase = loff_r * DIM
            for c0 in range(0, DIM, LANES):
                e = eb[b][r, pl.ds(c0, LANES)]
                p = pb[b][pl.ds(pbase + c0, LANES)]
                eb[b][r, pl.ds(c0, LANES)] = e * SCALE + p
            return carry

        lax.fori_loop(0, LANES, row_fn, 0)

        r0 = (gbase + ch) * LANES
        pltpu.async_copy(eb[b], out_hbm.at[pl.ds(r0, LANES)], ss[b])

        @pl.when(ch + 2 < CPW)
        def _():
            start_meta(ch + 2, b)

    start_meta(0, 0)
    start_meta(1, 1)
    start_emb(0, 0)
    wait_meta(0, 0)
    start_window(0)

    def pair_fn(pair, carry):
        step(2 * pair, 0)
        step(2 * pair + 1, 1)
        return carry

    lax.fori_loop(0, CPW // 2, pair_fn, 0)
    wait_store(CPW - 1, 1)


@functools.cache
def _sc_apply():
    return pl.kernel(
        _sc_body,
        mesh=plsc.VectorSubcoreMesh(core_axis_name="c", subcore_axis_name="s"),
        compiler_params=pltpu.CompilerParams(needs_layout_passes=False),
        out_type=jax.ShapeDtypeStruct((N, DIM), jnp.float32),
        scratch_types=[
            pltpu.VMEM((LANES, DIM), jnp.float32),
            pltpu.VMEM((LANES, DIM), jnp.float32),
            pltpu.VMEM((PBUF_ROWS * DIM,), jnp.float32),
            pltpu.VMEM((PBUF_ROWS * DIM,), jnp.float32),
            pltpu.VMEM((2 * LANES,), jnp.int32),
            pltpu.VMEM((2 * LANES,), jnp.int32),
            pltpu.SemaphoreType.DMA, pltpu.SemaphoreType.DMA,
            pltpu.SemaphoreType.DMA, pltpu.SemaphoreType.DMA,
            pltpu.SemaphoreType.DMA, pltpu.SemaphoreType.DMA,
            pltpu.SemaphoreType.DMA, pltpu.SemaphoreType.DMA,
        ],
    )


def _tc_body(out_in_ref, emb_ref, pidx_ref, pe_ref, out_ref):
    del out_in_ref
    pidx = pidx_ref[0, 0, :].reshape(BLK, 1)
    iota = lax.broadcasted_iota(jnp.int32, (BLK, PE_TC_ROWS), 1)
    onehot = (pidx == iota).astype(jnp.bfloat16)
    pe_sel = lax.dot_general(onehot, pe_ref[...], (((1,), (0,)), ((), ())),
                             preferred_element_type=jnp.float32)
    out_ref[...] = emb_ref[...] * SCALE + pe_sel


@functools.cache
def _tc_apply():
    return pl.pallas_call(
        _tc_body,
        grid=(TC_BLOCKS,),
        in_specs=[
            pl.BlockSpec((8, 128), lambda j: (0, 0)),
            pl.BlockSpec((BLK, DIM), lambda j: (SC_BLOCKS + j, 0)),
            pl.BlockSpec((1, 1, BLK), lambda j: (SC_BLOCKS + j, 0, 0)),
            pl.BlockSpec((PE_TC_ROWS, DIM), lambda j: (0, 0)),
        ],
        out_specs=pl.BlockSpec((BLK, DIM), lambda j: (SC_BLOCKS + j, 0)),
        out_shape=jax.ShapeDtypeStruct((N, DIM), jnp.float32),
        input_output_aliases={0: 0},
    )


def kernel(emb, position_mask):
    mask_i32 = position_mask.reshape(NUM_CHUNKS, LANES).astype(jnp.int32)
    meta, pidx = _chunk_meta(mask_i32)
    emb2d = emb.reshape(N, DIM)
    out_sc = _sc_apply()(emb2d, meta.reshape(-1),
                         jnp.asarray(_PE).reshape(-1))
    out_tc = _tc_apply()(
        out_sc,
        emb2d,
        pidx.reshape(N // BLK, 1, BLK),
        jnp.asarray(_PE[:PE_TC_ROWS].astype(np.dtype(jnp.bfloat16))),
    )
    return out_tc.reshape(emb.shape)


# speedup vs baseline: 7.1400x; 1.0837x over previous
import functools
import math

import jax
import jax.numpy as jnp
import numpy as np
from jax import lax
from jax.experimental import pallas as pl
from jax.experimental.pallas import tpu as pltpu
from jax.experimental.pallas import tpu_sc as plsc

MAX_LEN = 5000
DIM = 1024
N = 16384
B = 8
SCALE = math.sqrt(DIM)

LANES = 16
NUM_CHUNKS = N // LANES
NUM_CORES = 2
NUM_SUBCORES = 16
NUM_WORKERS = NUM_CORES * NUM_SUBCORES
CPW = 6
STATIC_ROWS = LANES
WIN_ROW = STATIC_ROWS
ZROW = 2 * LANES
PBUF_ROWS = 2 * LANES + 1
PE_ROWS = 2047

BLK = 512
SC_BLOCKS = 6
TC_BLOCKS = N // BLK - SC_BLOCKS
SC_CHUNKS = SC_BLOCKS * BLK // LANES
PE_TC_ROWS = 2048


def _pe_table() -> np.ndarray:
    pe = np.zeros((MAX_LEN + 1, DIM), dtype=np.float32)
    position = np.arange(0, MAX_LEN, dtype=np.float32)[:, None]
    div_term = np.exp(
        np.arange(0, DIM, 2, dtype=np.float32) * -(math.log(10000.0) / DIM))
    pe[:MAX_LEN, 0::2] = np.sin(position * div_term)
    pe[:MAX_LEN, 1::2] = np.cos(position * div_term)
    return pe


_PE = _pe_table()


def _meta_body(mask_ref, out_ref, pidx_ref):
    m = mask_ref[...]
    x = m
    for sh in (1, 2, 4, 8):
        x = x + jnp.concatenate(
            [jnp.zeros((NUM_CHUNKS, sh), jnp.int32), x[:, :-sh]], axis=1)
    rowm = x[:, LANES - 1:LANES]
    y = rowm
    for sh in (1, 2, 4, 8, 16, 32, 64, 128, 256, 512):
        y = y + jnp.concatenate(
            [jnp.zeros((sh, 1), jnp.int32), y[:-sh, :]], axis=0)
    k0 = y - rowm
    rank = k0 + x - 1
    col = lax.broadcasted_iota(jnp.int32, (NUM_CHUNKS, LANES), 1)
    bmod = col & 7
    cums, starts = [], []
    running = jnp.zeros((), jnp.int32)
    for b in range(B):
        sl_b = jnp.sum(jnp.where(bmod == b, m, 0))
        starts.append(running)
        running = running + sl_b
        cums.append(running)
    batchv = jnp.zeros((NUM_CHUNKS, LANES), jnp.int32)
    batch0 = jnp.zeros((NUM_CHUNKS, 1), jnp.int32)
    for b in range(B):
        batchv = batchv + (rank >= cums[b]).astype(jnp.int32)
        batch0 = batch0 + (k0 >= cums[b]).astype(jnp.int32)
    batchv = jnp.minimum(batchv, B - 1)
    batch0 = jnp.minimum(batch0, B - 1)
    startv = jnp.zeros((NUM_CHUNKS, LANES), jnp.int32)
    start0 = jnp.zeros((NUM_CHUNKS, 1), jnp.int32)
    for b in range(B):
        startv = startv + jnp.where(batchv == b, starts[b], 0)
        start0 = start0 + jnp.where(batch0 == b, starts[b], 0)
    pos = rank - startv
    w0 = jnp.clip(k0 - start0, 1, PE_ROWS + 1 - LANES)
    active = m > 0
    loff = jnp.where(active,
                     jnp.where(pos >= w0, pos - w0 + WIN_ROW, pos),
                     ZROW)
    out_ref[...] = jnp.concatenate(
        [jnp.broadcast_to(w0, (NUM_CHUNKS, LANES)), loff], axis=1)
    pidx_ref[...] = jnp.where(active, pos, PE_TC_ROWS)


def _chunk_meta(mask_i32):
    return pl.pallas_call(
        _meta_body,
        out_shape=(
            jax.ShapeDtypeStruct((NUM_CHUNKS, 2 * LANES), jnp.int32),
            jax.ShapeDtypeStruct((NUM_CHUNKS, LANES), jnp.int32),
        ),
    )(mask_i32)


CHUNK_ELEMS = LANES * DIM
UNROLL = 8


def _sc_body(emb_hbm, meta_hbm, pe_hbm, out_hbm,
             eb0, eb1, pb0, pb1, mb0, mb1,
             es0, es1, gs0, gs1, ss0, ss1, ms0, ms1):
    eb, pb, mb = (eb0, eb1), (pb0, pb1), (mb0, mb1)
    es, gs, ss, ms = (es0, es1), (gs0, gs1), (ss0, ss1), (ms0, ms1)
    wid = lax.axis_index("s") * NUM_CORES + lax.axis_index("c")
    gbase = wid * CPW

    for i in (0, 1):
        pltpu.sync_copy(pe_hbm.at[pl.ds(0, STATIC_ROWS * DIM)],
                        pb[i].at[pl.ds(0, STATIC_ROWS * DIM)])
        pltpu.sync_copy(pe_hbm.at[pl.ds(MAX_LEN * DIM, DIM)],
                        pb[i].at[pl.ds(ZROW * DIM, DIM)])

    def start_meta(ch, b):
        pltpu.async_copy(meta_hbm.at[pl.ds((gbase + ch) * 2 * LANES,
                                           2 * LANES)], mb[b], ms[b])

    def wait_meta(ch, b):
        pltpu.make_async_copy(meta_hbm.at[pl.ds(0, 2 * LANES)],
                              mb[b], ms[b]).wait()

    def start_emb(ch, b):
        r0 = (gbase + ch) * LANES
        pltpu.async_copy(emb_hbm.at[pl.ds(r0, LANES)], eb[b], es[b])

    def wait_emb(b):
        pltpu.make_async_copy(emb_hbm.at[pl.ds(0, LANES)],
                              eb[b], es[b]).wait()

    def start_window(b):
        w0 = jnp.max(mb[b][pl.ds(0, LANES)])
        pltpu.async_copy(pe_hbm.at[pl.ds(w0 * DIM, LANES * DIM)],
                         pb[b].at[pl.ds(WIN_ROW * DIM, LANES * DIM)], gs[b])

    def wait_window(b):
        pltpu.make_async_copy(pe_hbm.at[pl.ds(0, LANES * DIM)],
                              pb[b].at[pl.ds(WIN_ROW * DIM, LANES * DIM)],
                              gs[b]).wait()

    def wait_store(ch, b):
        r0 = (gbase + ch) * LANES
        pltpu.make_async_copy(eb[b], out_hbm.at[pl.ds(r0, LANES)],
                              ss[b]).wait()

    def step(ch, b):
        nb = 1 - b

        @pl.when(ch >= 1)
        def _():
            wait_store(ch - 1, nb)

        @pl.when(ch + 1 < CPW)
        def _():
            start_emb(ch + 1, nb)
            wait_meta(ch + 1, nb)
            start_window(nb)

        wait_emb(b)
        wait_window(b)

        lvec = mb[b][pl.ds(LANES, LANES)]
        riota = lax.broadcasted_iota(jnp.int32, (LANES,), 0)

        def row_fn(r, carry):
            loff_r = jnp.max(jnp.where(riota == r, lvec, 0))
            pbase = loff_r * DIM
            for c0 in range(0, DIM, LANES):
                e = eb[b][r, pl.ds(c0, LANES)]
                p = pb[b][pl.ds(pbase + c0, LANES)]
                eb[b][r, pl.ds(c0, LANES)] = e * SCALE + p
            return carry

        lax.fori_loop(0, LANES, row_fn, 0)

        r0 = (gbase + ch) * LANES
        pltpu.async_copy(eb[b], out_hbm.at[pl.ds(r0, LANES)], ss[b])

        @pl.when(ch + 2 < CPW)
        def _():
            start_meta(ch + 2, b)

    start_meta(0, 0)
    start_meta(1, 1)
    start_emb(0, 0)
    wait_meta(0, 0)
    start_window(0)

    def pair_fn(pair, carry):
        step(2 * pair, 0)
        step(2 * pair + 1, 1)
        return carry

    lax.fori_loop(0, CPW // 2, pair_fn, 0)
    wait_store(CPW - 1, 1)


@functools.cache
def _sc_apply():
    return pl.kernel(
        _sc_body,
        mesh=plsc.VectorSubcoreMesh(core_axis_name="c", subcore_axis_name="s"),
        compiler_params=pltpu.CompilerParams(needs_layout_passes=False),
        out_type=jax.ShapeDtypeStruct((N, DIM), jnp.float32),
        scratch_types=[
            pltpu.VMEM((LANES, DIM), jnp.float32),
            pltpu.VMEM((LANES, DIM), jnp.float32),
            pltpu.VMEM((PBUF_ROWS * DIM,), jnp.float32),
            pltpu.VMEM((PBUF_ROWS * DIM,), jnp.float32),
            pltpu.VMEM((2 * LANES,), jnp.int32),
            pltpu.VMEM((2 * LANES,), jnp.int32),
            pltpu.SemaphoreType.DMA, pltpu.SemaphoreType.DMA,
            pltpu.SemaphoreType.DMA, pltpu.SemaphoreType.DMA,
            pltpu.SemaphoreType.DMA, pltpu.SemaphoreType.DMA,
            pltpu.SemaphoreType.DMA, pltpu.SemaphoreType.DMA,
        ],
    )


def _tc_body(out_in_ref, emb_ref, pidx_ref, pe_ref, out_ref):
    del out_in_ref
    pidx = pidx_ref[0, 0, :].reshape(BLK, 1)
    iota = lax.broadcasted_iota(jnp.int32, (BLK, PE_TC_ROWS), 1)
    onehot = (pidx == iota).astype(jnp.bfloat16)
    pe_sel = lax.dot_general(onehot, pe_ref[...], (((1,), (0,)), ((), ())),
                             preferred_element_type=jnp.float32)
    out_ref[...] = emb_ref[...] * SCALE + pe_sel


@functools.cache
def _tc_apply():
    return pl.pallas_call(
        _tc_body,
        grid=(TC_BLOCKS,),
        in_specs=[
            pl.BlockSpec((8, 128), lambda j: (0, 0)),
            pl.BlockSpec((BLK, DIM), lambda j: (SC_BLOCKS + j, 0)),
            pl.BlockSpec((1, 1, BLK), lambda j: (SC_BLOCKS + j, 0, 0)),
            pl.BlockSpec((PE_TC_ROWS, DIM), lambda j: (0, 0)),
        ],
        out_specs=pl.BlockSpec((BLK, DIM), lambda j: (SC_BLOCKS + j, 0)),
        out_shape=jax.ShapeDtypeStruct((N, DIM), jnp.float32),
        input_output_aliases={0: 0},
    )


def kernel(emb, position_mask):
    mask_i32 = position_mask.reshape(NUM_CHUNKS, LANES).astype(jnp.int32)
    meta, pidx = _chunk_meta(mask_i32)
    emb2d = emb.reshape(N, DIM)
    out_sc = _sc_apply()(emb2d, meta.reshape(-1),
                         jnp.asarray(_PE).reshape(-1))
    out_tc = _tc_apply()(
        out_sc,
        emb2d,
        pidx.reshape(N // BLK, 1, BLK),
        jnp.asarray(_PE[:PE_TC_ROWS].astype(np.dtype(jnp.bfloat16))),
    )
    return out_tc.reshape(emb.shape)


# speedup vs baseline: 7.4731x; 1.0467x over previous
import functools
import math

import jax
import jax.numpy as jnp
import numpy as np
from jax import lax
from jax.experimental import pallas as pl
from jax.experimental.pallas import tpu as pltpu
from jax.experimental.pallas import tpu_sc as plsc

MAX_LEN = 5000
DIM = 1024
N = 16384
B = 8
SCALE = math.sqrt(DIM)

LANES = 16
NUM_CHUNKS = N // LANES
NUM_CORES = 2
NUM_SUBCORES = 16
NUM_WORKERS = NUM_CORES * NUM_SUBCORES
CPW = 4
STATIC_ROWS = LANES
WIN_ROW = STATIC_ROWS
ZROW = 2 * LANES
PBUF_ROWS = 2 * LANES + 1
PE_ROWS = 2047

BLK = 512
SC_BLOCKS = 4
TC_BLOCKS = N // BLK - SC_BLOCKS
SC_CHUNKS = SC_BLOCKS * BLK // LANES
PE_TC_ROWS = 2048


def _pe_table() -> np.ndarray:
    pe = np.zeros((MAX_LEN + 1, DIM), dtype=np.float32)
    position = np.arange(0, MAX_LEN, dtype=np.float32)[:, None]
    div_term = np.exp(
        np.arange(0, DIM, 2, dtype=np.float32) * -(math.log(10000.0) / DIM))
    pe[:MAX_LEN, 0::2] = np.sin(position * div_term)
    pe[:MAX_LEN, 1::2] = np.cos(position * div_term)
    return pe


_PE = _pe_table()


def _meta_body(mask_ref, out_ref, pidx_ref):
    m = mask_ref[...]
    x = m
    for sh in (1, 2, 4, 8):
        x = x + jnp.concatenate(
            [jnp.zeros((NUM_CHUNKS, sh), jnp.int32), x[:, :-sh]], axis=1)
    rowm = x[:, LANES - 1:LANES]
    y = rowm
    for sh in (1, 2, 4, 8, 16, 32, 64, 128, 256, 512):
        y = y + jnp.concatenate(
            [jnp.zeros((sh, 1), jnp.int32), y[:-sh, :]], axis=0)
    k0 = y - rowm
    rank = k0 + x - 1
    col = lax.broadcasted_iota(jnp.int32, (NUM_CHUNKS, LANES), 1)
    bmod = col & 7
    cums, starts = [], []
    running = jnp.zeros((), jnp.int32)
    for b in range(B):
        sl_b = jnp.sum(jnp.where(bmod == b, m, 0))
        starts.append(running)
        running = running + sl_b
        cums.append(running)
    batchv = jnp.zeros((NUM_CHUNKS, LANES), jnp.int32)
    batch0 = jnp.zeros((NUM_CHUNKS, 1), jnp.int32)
    for b in range(B):
        batchv = batchv + (rank >= cums[b]).astype(jnp.int32)
        batch0 = batch0 + (k0 >= cums[b]).astype(jnp.int32)
    batchv = jnp.minimum(batchv, B - 1)
    batch0 = jnp.minimum(batch0, B - 1)
    startv = jnp.zeros((NUM_CHUNKS, LANES), jnp.int32)
    start0 = jnp.zeros((NUM_CHUNKS, 1), jnp.int32)
    for b in range(B):
        startv = startv + jnp.where(batchv == b, starts[b], 0)
        start0 = start0 + jnp.where(batch0 == b, starts[b], 0)
    pos = rank - startv
    w0 = jnp.clip(k0 - start0, 1, PE_ROWS + 1 - LANES)
    active = m > 0
    loff = jnp.where(active,
                     jnp.where(pos >= w0, pos - w0 + WIN_ROW, pos),
                     ZROW)
    out_ref[...] = jnp.concatenate(
        [jnp.broadcast_to(w0, (NUM_CHUNKS, LANES)), loff], axis=1)
    pidx_ref[...] = jnp.where(active, pos, PE_TC_ROWS)


def _chunk_meta(mask_i32):
    return pl.pallas_call(
        _meta_body,
        out_shape=(
            jax.ShapeDtypeStruct((NUM_CHUNKS, 2 * LANES), jnp.int32),
            jax.ShapeDtypeStruct((NUM_CHUNKS, LANES), jnp.int32),
        ),
    )(mask_i32)


CHUNK_ELEMS = LANES * DIM
UNROLL = 8


def _sc_body(emb_hbm, meta_hbm, pe_hbm, out_hbm,
             eb0, eb1, pb0, pb1, mb0, mb1,
             es0, es1, gs0, gs1, ss0, ss1, ms0, ms1):
    eb, pb, mb = (eb0, eb1), (pb0, pb1), (mb0, mb1)
    es, gs, ss, ms = (es0, es1), (gs0, gs1), (ss0, ss1), (ms0, ms1)
    wid = lax.axis_index("s") * NUM_CORES + lax.axis_index("c")
    gbase = wid * CPW

    for i in (0, 1):
        pltpu.sync_copy(pe_hbm.at[pl.ds(0, STATIC_ROWS * DIM)],
                        pb[i].at[pl.ds(0, STATIC_ROWS * DIM)])
        pltpu.sync_copy(pe_hbm.at[pl.ds(MAX_LEN * DIM, DIM)],
                        pb[i].at[pl.ds(ZROW * DIM, DIM)])

    def start_meta(ch, b):
        pltpu.async_copy(meta_hbm.at[pl.ds((gbase + ch) * 2 * LANES,
                                           2 * LANES)], mb[b], ms[b])

    def wait_meta(ch, b):
        pltpu.make_async_copy(meta_hbm.at[pl.ds(0, 2 * LANES)],
                              mb[b], ms[b]).wait()

    def start_emb(ch, b):
        r0 = (gbase + ch) * LANES
        pltpu.async_copy(emb_hbm.at[pl.ds(r0, LANES)], eb[b], es[b])

    def wait_emb(b):
        pltpu.make_async_copy(emb_hbm.at[pl.ds(0, LANES)],
                              eb[b], es[b]).wait()

    def start_window(b):
        w0 = jnp.max(mb[b][pl.ds(0, LANES)])
        pltpu.async_copy(pe_hbm.at[pl.ds(w0 * DIM, LANES * DIM)],
                         pb[b].at[pl.ds(WIN_ROW * DIM, LANES * DIM)], gs[b])

    def wait_window(b):
        pltpu.make_async_copy(pe_hbm.at[pl.ds(0, LANES * DIM)],
                              pb[b].at[pl.ds(WIN_ROW * DIM, LANES * DIM)],
                              gs[b]).wait()

    def wait_store(ch, b):
        r0 = (gbase + ch) * LANES
        pltpu.make_async_copy(eb[b], out_hbm.at[pl.ds(r0, LANES)],
                              ss[b]).wait()

    def step(ch, b):
        nb = 1 - b

        @pl.when(ch >= 1)
        def _():
            wait_store(ch - 1, nb)

        @pl.when(ch + 1 < CPW)
        def _():
            start_emb(ch + 1, nb)
            wait_meta(ch + 1, nb)
            start_window(nb)

        wait_emb(b)
        wait_window(b)

        lvec = mb[b][pl.ds(LANES, LANES)]
        riota = lax.broadcasted_iota(jnp.int32, (LANES,), 0)

        def row_fn(r, carry):
            loff_r = jnp.max(jnp.where(riota == r, lvec, 0))
            pbase = loff_r * DIM
            for c0 in range(0, DIM, LANES):
                e = eb[b][r, pl.ds(c0, LANES)]
                p = pb[b][pl.ds(pbase + c0, LANES)]
                eb[b][r, pl.ds(c0, LANES)] = e * SCALE + p
            return carry

        lax.fori_loop(0, LANES, row_fn, 0)

        r0 = (gbase + ch) * LANES
        pltpu.async_copy(eb[b], out_hbm.at[pl.ds(r0, LANES)], ss[b])

        @pl.when(ch + 2 < CPW)
        def _():
            start_meta(ch + 2, b)

    start_meta(0, 0)
    start_meta(1, 1)
    start_emb(0, 0)
    wait_meta(0, 0)
    start_window(0)

    def pair_fn(pair, carry):
        step(2 * pair, 0)
        step(2 * pair + 1, 1)
        return carry

    lax.fori_loop(0, CPW // 2, pair_fn, 0)
    wait_store(CPW - 1, 1)


@functools.cache
def _sc_apply():
    return pl.kernel(
        _sc_body,
        mesh=plsc.VectorSubcoreMesh(core_axis_name="c", subcore_axis_name="s"),
        compiler_params=pltpu.CompilerParams(needs_layout_passes=False),
        out_type=jax.ShapeDtypeStruct((N, DIM), jnp.float32),
        scratch_types=[
            pltpu.VMEM((LANES, DIM), jnp.float32),
            pltpu.VMEM((LANES, DIM), jnp.float32),
            pltpu.VMEM((PBUF_ROWS * DIM,), jnp.float32),
            pltpu.VMEM((PBUF_ROWS * DIM,), jnp.float32),
            pltpu.VMEM((2 * LANES,), jnp.int32),
            pltpu.VMEM((2 * LANES,), jnp.int32),
            pltpu.SemaphoreType.DMA, pltpu.SemaphoreType.DMA,
            pltpu.SemaphoreType.DMA, pltpu.SemaphoreType.DMA,
            pltpu.SemaphoreType.DMA, pltpu.SemaphoreType.DMA,
            pltpu.SemaphoreType.DMA, pltpu.SemaphoreType.DMA,
        ],
    )


def _tc_body(out_in_ref, emb_ref, pidx_ref, pe_ref, out_ref):
    del out_in_ref
    pidx = pidx_ref[0, 0, :].reshape(BLK, 1)
    iota = lax.broadcasted_iota(jnp.int32, (BLK, PE_TC_ROWS), 1)
    onehot = (pidx == iota).astype(jnp.bfloat16)
    pe_sel = lax.dot_general(onehot, pe_ref[...], (((1,), (0,)), ((), ())),
                             preferred_element_type=jnp.float32)
    out_ref[...] = emb_ref[...] * SCALE + pe_sel


@functools.cache
def _tc_apply():
    return pl.pallas_call(
        _tc_body,
        grid=(TC_BLOCKS,),
        in_specs=[
            pl.BlockSpec((8, 128), lambda j: (0, 0)),
            pl.BlockSpec((BLK, DIM), lambda j: (SC_BLOCKS + j, 0)),
            pl.BlockSpec((1, 1, BLK), lambda j: (SC_BLOCKS + j, 0, 0)),
            pl.BlockSpec((PE_TC_ROWS, DIM), lambda j: (0, 0)),
        ],
        out_specs=pl.BlockSpec((BLK, DIM), lambda j: (SC_BLOCKS + j, 0)),
        out_shape=jax.ShapeDtypeStruct((N, DIM), jnp.float32),
        input_output_aliases={0: 0},
    )


def kernel(emb, position_mask):
    mask_i32 = position_mask.reshape(NUM_CHUNKS, LANES).astype(jnp.int32)
    meta, pidx = _chunk_meta(mask_i32)
    emb2d = emb.reshape(N, DIM)
    out_sc = _sc_apply()(emb2d, meta.reshape(-1),
                         jnp.asarray(_PE).reshape(-1))
    out_tc = _tc_apply()(
        out_sc,
        emb2d,
        pidx.reshape(N // BLK, 1, BLK),
        jnp.asarray(_PE[:PE_TC_ROWS].astype(np.dtype(jnp.bfloat16))),
    )
    return out_tc.reshape(emb.shape)
